# baseline (speedup 1.0000x reference)
import functools

import jax
import jax.numpy as jnp
from jax import lax
from jax.experimental import pallas as pl
from jax.experimental.pallas import tpu as pltpu
from jax.experimental.pallas import tpu_sc as plsc

NSC = 2
NTILE = 16
NW = NSC * NTILE
CH = 125
DEGW = 16


def _edge_agg_kernel(n_acc, width, n_chunks, mesh):
    assert n_chunks % 4 == 0
    hchunks = n_chunks // 2

    @functools.partial(
        pl.kernel,
        out_type=jax.ShapeDtypeStruct((NSC, n_acc, width), jnp.float32),
        mesh=mesh,
        scratch_types=[
            pltpu.VMEM((hchunks, CH), jnp.int32),
            pltpu.VMEM((hchunks, CH), jnp.int32),
            pltpu.VMEM((CH, width), jnp.float32),
            pltpu.VMEM((CH, width), jnp.float32),
            pltpu.VMEM_SHARED((n_acc, width), jnp.float32),
            pltpu.SemaphoreType.DMA,
            pltpu.SemaphoreType.DMA,
        ],
    )
    def k(table, eidx, zeros_blk, out, src_v, dst_v, rows_a,
          rows_b, acc, sem_a, sem_b):
        c = lax.axis_index("c")
        s = lax.axis_index("s")
        wid = c * NTILE + s
        zrows = n_acc // NTILE
        pltpu.sync_copy(zeros_blk, acc.at[pl.ds(s * zrows, zrows)])
        plsc.subcore_barrier()

        for h in range(2):
            pltpu.sync_copy(eidx.at[0, wid, pl.ds(h * hchunks, hchunks)],
                            src_v)
            pltpu.sync_copy(eidx.at[1, wid, pl.ds(h * hchunks, hchunks)],
                            dst_v)
            pltpu.async_copy(table.at[src_v.at[0]], rows_a, sem_a)

            def body(jj, carry):
                j0 = 2 * jj
                j1 = j0 + 1
                pltpu.async_copy(table.at[src_v.at[j1]], rows_b, sem_b)
                pltpu.make_async_copy(table.at[src_v.at[j0]], rows_a,
                                      sem_a).wait()
                pltpu.sync_copy(rows_a, acc.at[dst_v.at[j0]], add=True)
                jn = jnp.minimum(j0 + 2, hchunks - 1)
                pltpu.async_copy(table.at[src_v.at[jn]], rows_a, sem_a)
                pltpu.make_async_copy(table.at[src_v.at[j1]], rows_b,
                                      sem_b).wait()
                pltpu.sync_copy(rows_b, acc.at[dst_v.at[j1]], add=True)
                return carry

            lax.fori_loop(0, hchunks // 2, body, 0)
            pltpu.make_async_copy(table.at[src_v.at[0]], rows_a, sem_a).wait()

        plsc.subcore_barrier()
        pltpu.sync_copy(acc.at[pl.ds(s * zrows, zrows)],
                        out.at[c, pl.ds(s * zrows, zrows)])

    return k


def _degree_kernel(n_acc, n_chunks, mesh):
    @functools.partial(
        pl.kernel,
        out_type=jax.ShapeDtypeStruct((NSC, 2, n_acc, DEGW), jnp.float32),
        mesh=mesh,
        scratch_types=[
            pltpu.VMEM((n_chunks, CH), jnp.int32),
            pltpu.VMEM((n_chunks, CH), jnp.int32),
            pltpu.VMEM((CH, DEGW), jnp.float32),
            pltpu.VMEM_SHARED((n_acc, DEGW), jnp.float32),
            pltpu.VMEM_SHARED((n_acc, DEGW), jnp.float32),
            pltpu.SemaphoreType.DMA,
            pltpu.SemaphoreType.DMA,
        ],
    )
    def k(eidx, ones_blk, zeros_blk, out, src_v, dst_v, ones_v,
          acc_s, acc_d, sem_s, sem_d):
        c = lax.axis_index("c")
        s = lax.axis_index("s")
        wid = c * NTILE + s
        zrows = n_acc // NTILE
        pltpu.sync_copy(zeros_blk, acc_s.at[pl.ds(s * zrows, zrows)])
        pltpu.sync_copy(zeros_blk, acc_d.at[pl.ds(s * zrows, zrows)])
        pltpu.sync_copy(ones_blk, ones_v)
        pltpu.sync_copy(eidx.at[0, wid], src_v)
        pltpu.sync_copy(eidx.at[1, wid], dst_v)
        plsc.subcore_barrier()

        def body(j, carry):
            pltpu.async_copy(ones_v, acc_s.at[src_v.at[j]], sem_s, add=True)
            pltpu.async_copy(ones_v, acc_d.at[dst_v.at[j]], sem_d, add=True)

            @pl.when(j >= 1)
            def _():
                pltpu.make_async_copy(ones_v, acc_s.at[src_v.at[j]],
                                      sem_s).wait()
                pltpu.make_async_copy(ones_v, acc_d.at[dst_v.at[j]],
                                      sem_d).wait()

            return carry

        lax.fori_loop(0, n_chunks, body, 0)
        pltpu.make_async_copy(ones_v, acc_s.at[src_v.at[0]], sem_s).wait()
        pltpu.make_async_copy(ones_v, acc_d.at[dst_v.at[0]], sem_d).wait()
        plsc.subcore_barrier()
        pltpu.sync_copy(acc_s.at[pl.ds(s * zrows, zrows)],
                        out.at[c, 0, pl.ds(s * zrows, zrows)])
        pltpu.sync_copy(acc_d.at[pl.ds(s * zrows, zrows)],
                        out.at[c, 1, pl.ds(s * zrows, zrows)])

    return k


def _norms_hs_body(deg_ref, feats_ref, hs_ref, on_ref, in_ref):
    dsrc = deg_ref[0, 0, :, :1] + deg_ref[1, 0, :, :1]
    ddst = deg_ref[0, 1, :, :1] + deg_ref[1, 1, :, :1]
    onorm = lax.rsqrt(jnp.maximum(dsrc, 1.0))
    inorm = lax.rsqrt(jnp.maximum(ddst, 1.0))
    on_ref[...] = onorm
    in_ref[...] = inorm
    hs_ref[...] = feats_ref[...] * onorm


def _layer1_body(agg_ref, in_ref, on_ref, w1_ref, b1_ref, w23_ref, out_ref):
    a = (agg_ref[0] + agg_ref[1]) * in_ref[...]
    h = jnp.maximum(
        jnp.dot(a, w1_ref[...], preferred_element_type=jnp.float32)
        + b1_ref[...], 0.0)
    p = jnp.dot(h, w23_ref[...], preferred_element_type=jnp.float32)
    out_ref[...] = p * on_ref[...]


def _z_body(agg_ref, in_ref, noise_ref, b2_ref, b3_ref, z_ref):
    d = noise_ref.shape[1]
    q = (agg_ref[0] + agg_ref[1]) * in_ref[...]
    mu = q[:, :d] + b2_ref[...]
    ls = q[:, d:] + b3_ref[...]
    z_ref[...] = mu + noise_ref[...] * jnp.exp(ls)


def _decoder_body(zi_ref, zall_ref, out_ref):
    logits = lax.dot_general(zi_ref[...], zall_ref[...],
                             (((1,), (1,)), ((), ())),
                             preferred_element_type=jnp.float32)
    out_ref[...] = jax.nn.sigmoid(logits)


def kernel(feats, edge_index, noise, W1, b1, W2, b2, W3, b3):
    n = feats.shape[0]
    d_in = feats.shape[1]
    d_hid = W1.shape[1]
    d_out = W2.shape[1]
    e = edge_index.shape[1]

    n_chunks = -(-e // (NW * CH))
    ep = NW * CH * n_chunks
    n_acc = ((n + 1 + 127) // 128) * 128
    mesh = plsc.VectorSubcoreMesh(core_axis_name="c", subcore_axis_name="s")

    pad = ep - e
    if pad:
        src = edge_index[0].astype(jnp.int32)
        dst = edge_index[1].astype(jnp.int32)
        eidx = jnp.stack([
            jnp.concatenate([src, jnp.zeros((pad,), jnp.int32)]),
            jnp.concatenate([dst, jnp.full((pad,), n, jnp.int32)]),
        ]).reshape(2, NW, n_chunks, CH)
    else:
        eidx = edge_index.astype(jnp.int32).reshape(2, NW, n_chunks, CH)

    zrows = n_acc // NTILE
    zeros_deg = jnp.zeros((zrows, DEGW), jnp.float32)
    ones_deg = jnp.ones((CH, DEGW), jnp.float32)
    zeros_f = jnp.zeros((zrows, d_in), jnp.float32)

    deg = _degree_kernel(n_acc, n_chunks, mesh)(eidx, ones_deg, zeros_deg)

    rb = 1000
    grid1 = n // rb
    hs1, onorm, inorm = pl.pallas_call(
        _norms_hs_body,
        grid=(grid1,),
        in_specs=[
            pl.BlockSpec((NSC, 2, rb, DEGW), lambda i: (0, 0, i, 0)),
            pl.BlockSpec((rb, d_in), lambda i: (i, 0)),
        ],
        out_specs=[
            pl.BlockSpec((rb, d_in), lambda i: (i, 0)),
            pl.BlockSpec((rb, 1), lambda i: (i, 0)),
            pl.BlockSpec((rb, 1), lambda i: (i, 0)),
        ],
        out_shape=[
            jax.ShapeDtypeStruct((n, d_in), jnp.float32),
            jax.ShapeDtypeStruct((n, 1), jnp.float32),
            jax.ShapeDtypeStruct((n, 1), jnp.float32),
        ],
    )(deg, feats)

    agg1 = _edge_agg_kernel(n_acc, d_in, n_chunks, mesh)(hs1, eidx, zeros_f)

    w23 = jnp.concatenate([W2, W3], axis=1)
    hs2 = pl.pallas_call(
        _layer1_body,
        grid=(grid1,),
        in_specs=[
            pl.BlockSpec((NSC, rb, d_in), lambda i: (0, i, 0)),
            pl.BlockSpec((rb, 1), lambda i: (i, 0)),
            pl.BlockSpec((rb, 1), lambda i: (i, 0)),
            pl.BlockSpec((d_in, d_hid), lambda i: (0, 0)),
            pl.BlockSpec((1, d_hid), lambda i: (0, 0)),
            pl.BlockSpec((d_hid, 2 * d_out), lambda i: (0, 0)),
        ],
        out_specs=pl.BlockSpec((rb, 2 * d_out), lambda i: (i, 0)),
        out_shape=jax.ShapeDtypeStruct((n, 2 * d_out), jnp.float32),
    )(agg1, inorm, onorm, W1, b1.reshape(1, d_hid), w23)

    agg2 = _edge_agg_kernel(n_acc, 2 * d_out, n_chunks, mesh)(
        hs2, eidx, zeros_f[:, :2 * d_out])

    z = pl.pallas_call(
        _z_body,
        grid=(grid1,),
        in_specs=[
            pl.BlockSpec((NSC, rb, 2 * d_out), lambda i: (0, i, 0)),
            pl.BlockSpec((rb, 1), lambda i: (i, 0)),
            pl.BlockSpec((rb, d_out), lambda i: (i, 0)),
            pl.BlockSpec((1, d_out), lambda i: (0, 0)),
            pl.BlockSpec((1, d_out), lambda i: (0, 0)),
        ],
        out_specs=pl.BlockSpec((rb, d_out), lambda i: (i, 0)),
        out_shape=jax.ShapeDtypeStruct((n, d_out), jnp.float32),
    )(agg2, inorm, noise, b2.reshape(1, d_out), b3.reshape(1, d_out))

    rb2 = 400
    adj = pl.pallas_call(
        _decoder_body,
        grid=(n // rb2,),
        in_specs=[
            pl.BlockSpec((rb2, d_out), lambda i: (i, 0)),
            pl.BlockSpec((n, d_out), lambda i: (0, 0)),
        ],
        out_specs=pl.BlockSpec((rb2, n), lambda i: (i, 0)),
        out_shape=jax.ShapeDtypeStruct((n, n), jnp.float32),
    )(z, z)
    return adj


# speedup vs baseline: 1.0040x; 1.0040x over previous
import functools

import jax
import jax.numpy as jnp
from jax import lax
from jax.experimental import pallas as pl
from jax.experimental.pallas import tpu as pltpu
from jax.experimental.pallas import tpu_sc as plsc

NSC = 2
NTILE = 16
NW = NSC * NTILE
CH = 125
DEGW = 16


def _edge_agg_kernel(n_acc, width, n_chunks, mesh):
    assert n_chunks % 4 == 0
    hchunks = n_chunks // 2

    @functools.partial(
        pl.kernel,
        out_type=jax.ShapeDtypeStruct((NSC, n_acc, width), jnp.float32),
        mesh=mesh,
        scratch_types=[
            pltpu.VMEM((hchunks, CH), jnp.int32),
            pltpu.VMEM((hchunks, CH), jnp.int32),
            pltpu.VMEM((CH, width), jnp.float32),
            pltpu.VMEM((CH, width), jnp.float32),
            pltpu.VMEM_SHARED((n_acc, width), jnp.float32),
            pltpu.SemaphoreType.DMA,
            pltpu.SemaphoreType.DMA,
        ],
    )
    def k(table, eidx, zeros_blk, out, src_v, dst_v, rows_a,
          rows_b, acc, sem_a, sem_b):
        c = lax.axis_index("c")
        s = lax.axis_index("s")
        wid = c * NTILE + s
        zrows = n_acc // NTILE
        pltpu.async_copy(zeros_blk, acc.at[pl.ds(s * zrows, zrows)], sem_a)
        pltpu.async_copy(eidx.at[0, wid, pl.ds(0, hchunks)], src_v, sem_b)
        pltpu.async_copy(eidx.at[1, wid, pl.ds(0, hchunks)], dst_v, sem_b)
        pltpu.make_async_copy(zeros_blk, acc.at[pl.ds(s * zrows, zrows)],
                              sem_a).wait()
        pltpu.make_async_copy(eidx.at[0, wid, pl.ds(0, hchunks)], src_v,
                              sem_b).wait()
        pltpu.make_async_copy(eidx.at[1, wid, pl.ds(0, hchunks)], dst_v,
                              sem_b).wait()
        plsc.subcore_barrier()

        for h in range(2):
            if h:
                pltpu.sync_copy(eidx.at[0, wid, pl.ds(h * hchunks, hchunks)],
                                src_v)
                pltpu.sync_copy(eidx.at[1, wid, pl.ds(h * hchunks, hchunks)],
                                dst_v)
            pltpu.async_copy(table.at[src_v.at[0]], rows_a, sem_a)

            def body(jj, carry):
                j0 = 2 * jj
                j1 = j0 + 1
                pltpu.async_copy(table.at[src_v.at[j1]], rows_b, sem_b)
                pltpu.make_async_copy(table.at[src_v.at[j0]], rows_a,
                                      sem_a).wait()
                pltpu.sync_copy(rows_a, acc.at[dst_v.at[j0]], add=True)
                jn = jnp.minimum(j0 + 2, hchunks - 1)
                pltpu.async_copy(table.at[src_v.at[jn]], rows_a, sem_a)
                pltpu.make_async_copy(table.at[src_v.at[j1]], rows_b,
                                      sem_b).wait()
                pltpu.sync_copy(rows_b, acc.at[dst_v.at[j1]], add=True)
                return carry

            lax.fori_loop(0, hchunks // 2, body, 0)
            pltpu.make_async_copy(table.at[src_v.at[0]], rows_a, sem_a).wait()

        plsc.subcore_barrier()
        pltpu.sync_copy(acc.at[pl.ds(s * zrows, zrows)],
                        out.at[c, pl.ds(s * zrows, zrows)])

    return k


def _degree_kernel(n_acc, n_chunks, mesh):
    @functools.partial(
        pl.kernel,
        out_type=jax.ShapeDtypeStruct((NSC, 2, n_acc, DEGW), jnp.float32),
        mesh=mesh,
        scratch_types=[
            pltpu.VMEM((n_chunks, CH), jnp.int32),
            pltpu.VMEM((n_chunks, CH), jnp.int32),
            pltpu.VMEM((CH, DEGW), jnp.float32),
            pltpu.VMEM_SHARED((n_acc, DEGW), jnp.float32),
            pltpu.VMEM_SHARED((n_acc, DEGW), jnp.float32),
            pltpu.SemaphoreType.DMA,
            pltpu.SemaphoreType.DMA,
        ],
    )
    def k(eidx, ones_blk, zeros_blk, out, src_v, dst_v, ones_v,
          acc_s, acc_d, sem_s, sem_d):
        c = lax.axis_index("c")
        s = lax.axis_index("s")
        wid = c * NTILE + s
        zrows = n_acc // NTILE
        pltpu.sync_copy(zeros_blk, acc_s.at[pl.ds(s * zrows, zrows)])
        pltpu.sync_copy(zeros_blk, acc_d.at[pl.ds(s * zrows, zrows)])
        pltpu.sync_copy(ones_blk, ones_v)
        pltpu.sync_copy(eidx.at[0, wid], src_v)
        pltpu.sync_copy(eidx.at[1, wid], dst_v)
        plsc.subcore_barrier()

        def body(j, carry):
            pltpu.async_copy(ones_v, acc_s.at[src_v.at[j]], sem_s, add=True)
            pltpu.async_copy(ones_v, acc_d.at[dst_v.at[j]], sem_d, add=True)

            @pl.when(j >= 1)
            def _():
                pltpu.make_async_copy(ones_v, acc_s.at[src_v.at[j]],
                                      sem_s).wait()
                pltpu.make_async_copy(ones_v, acc_d.at[dst_v.at[j]],
                                      sem_d).wait()

            return carry

        lax.fori_loop(0, n_chunks, body, 0)
        pltpu.make_async_copy(ones_v, acc_s.at[src_v.at[0]], sem_s).wait()
        pltpu.make_async_copy(ones_v, acc_d.at[dst_v.at[0]], sem_d).wait()
        plsc.subcore_barrier()
        pltpu.sync_copy(acc_s.at[pl.ds(s * zrows, zrows)],
                        out.at[c, 0, pl.ds(s * zrows, zrows)])
        pltpu.sync_copy(acc_d.at[pl.ds(s * zrows, zrows)],
                        out.at[c, 1, pl.ds(s * zrows, zrows)])

    return k


def _norms_hs_body(deg_ref, feats_ref, hs_ref, on_ref, in_ref):
    dsrc = deg_ref[0, 0, :, :1] + deg_ref[1, 0, :, :1]
    ddst = deg_ref[0, 1, :, :1] + deg_ref[1, 1, :, :1]
    onorm = lax.rsqrt(jnp.maximum(dsrc, 1.0))
    inorm = lax.rsqrt(jnp.maximum(ddst, 1.0))
    on_ref[...] = onorm
    in_ref[...] = inorm
    hs_ref[...] = feats_ref[...] * onorm


def _layer1_body(agg_ref, in_ref, on_ref, w1_ref, b1_ref, w23_ref, out_ref):
    a = (agg_ref[0] + agg_ref[1]) * in_ref[...]
    h = jnp.maximum(
        jnp.dot(a, w1_ref[...], preferred_element_type=jnp.float32)
        + b1_ref[...], 0.0)
    p = jnp.dot(h, w23_ref[...], preferred_element_type=jnp.float32)
    out_ref[...] = p * on_ref[...]


def _z_body(agg_ref, in_ref, noise_ref, b2_ref, b3_ref, z_ref):
    d = noise_ref.shape[1]
    q = (agg_ref[0] + agg_ref[1]) * in_ref[...]
    mu = q[:, :d] + b2_ref[...]
    ls = q[:, d:] + b3_ref[...]
    z_ref[...] = mu + noise_ref[...] * jnp.exp(ls)


def _decoder_body(zi_ref, zall_ref, out_ref):
    logits = lax.dot_general(zi_ref[...], zall_ref[...],
                             (((1,), (1,)), ((), ())),
                             preferred_element_type=jnp.float32)
    out_ref[...] = 1.0 / (1.0 + jnp.exp(-logits))


def kernel(feats, edge_index, noise, W1, b1, W2, b2, W3, b3):
    n = feats.shape[0]
    d_in = feats.shape[1]
    d_hid = W1.shape[1]
    d_out = W2.shape[1]
    e = edge_index.shape[1]

    n_chunks = -(-e // (NW * CH))
    ep = NW * CH * n_chunks
    n_acc = ((n + 1 + 127) // 128) * 128
    mesh = plsc.VectorSubcoreMesh(core_axis_name="c", subcore_axis_name="s")

    pad = ep - e
    if pad:
        src = edge_index[0].astype(jnp.int32)
        dst = edge_index[1].astype(jnp.int32)
        eidx = jnp.stack([
            jnp.concatenate([src, jnp.zeros((pad,), jnp.int32)]),
            jnp.concatenate([dst, jnp.full((pad,), n, jnp.int32)]),
        ]).reshape(2, NW, n_chunks, CH)
    else:
        eidx = edge_index.astype(jnp.int32).reshape(2, NW, n_chunks, CH)

    zrows = n_acc // NTILE
    zeros_deg = jnp.zeros((zrows, DEGW), jnp.float32)
    ones_deg = jnp.ones((CH, DEGW), jnp.float32)
    zeros_f = jnp.zeros((zrows, d_in), jnp.float32)

    deg = _degree_kernel(n_acc, n_chunks, mesh)(eidx, ones_deg, zeros_deg)

    rb = 1000
    grid1 = n // rb
    hs1, onorm, inorm = pl.pallas_call(
        _norms_hs_body,
        grid=(grid1,),
        in_specs=[
            pl.BlockSpec((NSC, 2, rb, DEGW), lambda i: (0, 0, i, 0)),
            pl.BlockSpec((rb, d_in), lambda i: (i, 0)),
        ],
        out_specs=[
            pl.BlockSpec((rb, d_in), lambda i: (i, 0)),
            pl.BlockSpec((rb, 1), lambda i: (i, 0)),
            pl.BlockSpec((rb, 1), lambda i: (i, 0)),
        ],
        out_shape=[
            jax.ShapeDtypeStruct((n, d_in), jnp.float32),
            jax.ShapeDtypeStruct((n, 1), jnp.float32),
            jax.ShapeDtypeStruct((n, 1), jnp.float32),
        ],
    )(deg, feats)

    agg1 = _edge_agg_kernel(n_acc, d_in, n_chunks, mesh)(hs1, eidx, zeros_f)

    w23 = jnp.concatenate([W2, W3], axis=1)
    hs2 = pl.pallas_call(
        _layer1_body,
        grid=(grid1,),
        in_specs=[
            pl.BlockSpec((NSC, rb, d_in), lambda i: (0, i, 0)),
            pl.BlockSpec((rb, 1), lambda i: (i, 0)),
            pl.BlockSpec((rb, 1), lambda i: (i, 0)),
            pl.BlockSpec((d_in, d_hid), lambda i: (0, 0)),
            pl.BlockSpec((1, d_hid), lambda i: (0, 0)),
            pl.BlockSpec((d_hid, 2 * d_out), lambda i: (0, 0)),
        ],
        out_specs=pl.BlockSpec((rb, 2 * d_out), lambda i: (i, 0)),
        out_shape=jax.ShapeDtypeStruct((n, 2 * d_out), jnp.float32),
    )(agg1, inorm, onorm, W1, b1.reshape(1, d_hid), w23)

    agg2 = _edge_agg_kernel(n_acc, 2 * d_out, n_chunks, mesh)(
        hs2, eidx, zeros_f[:, :2 * d_out])

    z = pl.pallas_call(
        _z_body,
        grid=(grid1,),
        in_specs=[
            pl.BlockSpec((NSC, rb, 2 * d_out), lambda i: (0, i, 0)),
            pl.BlockSpec((rb, 1), lambda i: (i, 0)),
            pl.BlockSpec((rb, d_out), lambda i: (i, 0)),
            pl.BlockSpec((1, d_out), lambda i: (0, 0)),
            pl.BlockSpec((1, d_out), lambda i: (0, 0)),
        ],
        out_specs=pl.BlockSpec((rb, d_out), lambda i: (i, 0)),
        out_shape=jax.ShapeDtypeStruct((n, d_out), jnp.float32),
    )(agg2, inorm, noise, b2.reshape(1, d_out), b3.reshape(1, d_out))

    rb2 = 400
    adj = pl.pallas_call(
        _decoder_body,
        grid=(n // rb2,),
        in_specs=[
            pl.BlockSpec((rb2, d_out), lambda i: (i, 0)),
            pl.BlockSpec((n, d_out), lambda i: (0, 0)),
        ],
        out_specs=pl.BlockSpec((rb2, n), lambda i: (i, 0)),
        out_shape=jax.ShapeDtypeStruct((n, n), jnp.float32),
    )(z, z)
    return adj


# speedup vs baseline: 1.0172x; 1.0131x over previous
import functools

import jax
import jax.numpy as jnp
from jax import lax
from jax.experimental import pallas as pl
from jax.experimental.pallas import tpu as pltpu
from jax.experimental.pallas import tpu_sc as plsc

NSC = 2
NTILE = 16
NW = NSC * NTILE
CH = 125
DEGW = 16


def _edge_agg_kernel(n_acc, width, n_chunks, mesh):
    assert n_chunks % 4 == 0
    hchunks = n_chunks // 2

    @functools.partial(
        pl.kernel,
        out_type=jax.ShapeDtypeStruct((NSC, n_acc, width), jnp.float32),
        mesh=mesh,
        scratch_types=[
            pltpu.VMEM((hchunks, CH), jnp.int32),
            pltpu.VMEM((hchunks, CH), jnp.int32),
            pltpu.VMEM((CH, width), jnp.float32),
            pltpu.VMEM((CH, width), jnp.float32),
            pltpu.VMEM_SHARED((n_acc, width), jnp.float32),
            pltpu.SemaphoreType.DMA,
            pltpu.SemaphoreType.DMA,
        ],
    )
    def k(table, eidx, zeros_blk, out, src_v, dst_v, rows_a,
          rows_b, acc, sem_a, sem_b):
        c = lax.axis_index("c")
        s = lax.axis_index("s")
        wid = c * NTILE + s
        zrows = n_acc // NTILE
        pltpu.async_copy(zeros_blk, acc.at[pl.ds(s * zrows, zrows)], sem_a)
        pltpu.async_copy(eidx.at[0, wid, pl.ds(0, hchunks)], src_v, sem_b)
        pltpu.async_copy(eidx.at[1, wid, pl.ds(0, hchunks)], dst_v, sem_b)
        pltpu.make_async_copy(zeros_blk, acc.at[pl.ds(s * zrows, zrows)],
                              sem_a).wait()
        pltpu.make_async_copy(eidx.at[0, wid, pl.ds(0, hchunks)], src_v,
                              sem_b).wait()
        pltpu.make_async_copy(eidx.at[1, wid, pl.ds(0, hchunks)], dst_v,
                              sem_b).wait()
        plsc.subcore_barrier()

        for h in range(2):
            if h:
                pltpu.sync_copy(eidx.at[0, wid, pl.ds(h * hchunks, hchunks)],
                                src_v)
                pltpu.sync_copy(eidx.at[1, wid, pl.ds(h * hchunks, hchunks)],
                                dst_v)
            pltpu.async_copy(table.at[src_v.at[0]], rows_a, sem_a)

            def body(jj, carry):
                j0 = 2 * jj
                j1 = j0 + 1
                pltpu.async_copy(table.at[src_v.at[j1]], rows_b, sem_b)
                pltpu.make_async_copy(table.at[src_v.at[j0]], rows_a,
                                      sem_a).wait()
                pltpu.sync_copy(rows_a, acc.at[dst_v.at[j0]], add=True)
                jn = jnp.minimum(j0 + 2, hchunks - 1)
                pltpu.async_copy(table.at[src_v.at[jn]], rows_a, sem_a)
                pltpu.make_async_copy(table.at[src_v.at[j1]], rows_b,
                                      sem_b).wait()
                pltpu.sync_copy(rows_b, acc.at[dst_v.at[j1]], add=True)
                return carry

            lax.fori_loop(0, hchunks // 2, body, 0)
            pltpu.make_async_copy(table.at[src_v.at[0]], rows_a, sem_a).wait()

        plsc.subcore_barrier()
        pltpu.sync_copy(acc.at[pl.ds(s * zrows, zrows)],
                        out.at[c, pl.ds(s * zrows, zrows)])

    return k


def _degree_kernel(n_acc, n_chunks, mesh):
    @functools.partial(
        pl.kernel,
        out_type=jax.ShapeDtypeStruct((NSC, 2, n_acc, DEGW), jnp.float32),
        mesh=mesh,
        scratch_types=[
            pltpu.VMEM((n_chunks, CH), jnp.int32),
            pltpu.VMEM((n_chunks, CH), jnp.int32),
            pltpu.VMEM((CH, DEGW), jnp.float32),
            pltpu.VMEM_SHARED((n_acc, DEGW), jnp.float32),
            pltpu.VMEM_SHARED((n_acc, DEGW), jnp.float32),
            pltpu.SemaphoreType.DMA,
            pltpu.SemaphoreType.DMA,
        ],
    )
    def k(eidx, ones_blk, zeros_blk, out, src_v, dst_v, ones_v,
          acc_s, acc_d, sem_s, sem_d):
        c = lax.axis_index("c")
        s = lax.axis_index("s")
        wid = c * NTILE + s
        zrows = n_acc // NTILE
        pltpu.sync_copy(zeros_blk, acc_s.at[pl.ds(s * zrows, zrows)])
        pltpu.sync_copy(zeros_blk, acc_d.at[pl.ds(s * zrows, zrows)])
        pltpu.sync_copy(ones_blk, ones_v)
        pltpu.sync_copy(eidx.at[0, wid], src_v)
        pltpu.sync_copy(eidx.at[1, wid], dst_v)
        plsc.subcore_barrier()

        def body(j, carry):
            pltpu.async_copy(ones_v, acc_s.at[src_v.at[j]], sem_s, add=True)
            pltpu.async_copy(ones_v, acc_d.at[dst_v.at[j]], sem_d, add=True)

            @pl.when(j >= 1)
            def _():
                pltpu.make_async_copy(ones_v, acc_s.at[src_v.at[j]],
                                      sem_s).wait()
                pltpu.make_async_copy(ones_v, acc_d.at[dst_v.at[j]],
                                      sem_d).wait()

            return carry

        lax.fori_loop(0, n_chunks, body, 0)
        pltpu.make_async_copy(ones_v, acc_s.at[src_v.at[0]], sem_s).wait()
        pltpu.make_async_copy(ones_v, acc_d.at[dst_v.at[0]], sem_d).wait()
        plsc.subcore_barrier()
        pltpu.sync_copy(acc_s.at[pl.ds(s * zrows, zrows)],
                        out.at[c, 0, pl.ds(s * zrows, zrows)])
        pltpu.sync_copy(acc_d.at[pl.ds(s * zrows, zrows)],
                        out.at[c, 1, pl.ds(s * zrows, zrows)])

    return k


def _norms_hs_body(deg_ref, feats_ref, hs_ref, on_ref, in_ref):
    dsrc = deg_ref[0, 0, :, :1] + deg_ref[1, 0, :, :1]
    ddst = deg_ref[0, 1, :, :1] + deg_ref[1, 1, :, :1]
    onorm = lax.rsqrt(jnp.maximum(dsrc, 1.0))
    inorm = lax.rsqrt(jnp.maximum(ddst, 1.0))
    on_ref[...] = onorm
    in_ref[...] = inorm
    hs_ref[...] = feats_ref[...] * onorm


def _layer1_body(agg_ref, in_ref, on_ref, w1_ref, b1_ref, w23_ref, out_ref):
    a = (agg_ref[0] + agg_ref[1]) * in_ref[...]
    h = jnp.maximum(
        jnp.dot(a, w1_ref[...], preferred_element_type=jnp.float32)
        + b1_ref[...], 0.0)
    p = jnp.dot(h, w23_ref[...], preferred_element_type=jnp.float32)
    out_ref[...] = p * on_ref[...]


def _decoder_body(agg_ref, in_ref, noise_ref, b2_ref, b3_ref, out_ref, z_scr):
    @pl.when(pl.program_id(0) == 0)
    def _():
        d = noise_ref.shape[1]
        cb = 1000

        def zb(j, carry):
            r = pl.ds(j * cb, cb)
            q = (agg_ref[0, r, :] + agg_ref[1, r, :]) * in_ref[r, :]
            mu = q[:, :d] + b2_ref[...]
            ls = q[:, d:] + b3_ref[...]
            z_scr[r, :] = mu + noise_ref[r, :] * jnp.exp(ls)
            return carry

        lax.fori_loop(0, noise_ref.shape[0] // cb, zb, 0)

    rb2 = out_ref.shape[0]
    zi = z_scr[pl.ds(pl.program_id(0) * rb2, rb2), :]
    logits = lax.dot_general(zi, z_scr[...],
                             (((1,), (1,)), ((), ())),
                             preferred_element_type=jnp.float32)
    out_ref[...] = 1.0 / (1.0 + jnp.exp(-logits))


def kernel(feats, edge_index, noise, W1, b1, W2, b2, W3, b3):
    n = feats.shape[0]
    d_in = feats.shape[1]
    d_hid = W1.shape[1]
    d_out = W2.shape[1]
    e = edge_index.shape[1]

    n_chunks = -(-e // (NW * CH))
    ep = NW * CH * n_chunks
    n_acc = ((n + 1 + 127) // 128) * 128
    mesh = plsc.VectorSubcoreMesh(core_axis_name="c", subcore_axis_name="s")

    pad = ep - e
    if pad:
        src = edge_index[0].astype(jnp.int32)
        dst = edge_index[1].astype(jnp.int32)
        eidx = jnp.stack([
            jnp.concatenate([src, jnp.zeros((pad,), jnp.int32)]),
            jnp.concatenate([dst, jnp.full((pad,), n, jnp.int32)]),
        ]).reshape(2, NW, n_chunks, CH)
    else:
        eidx = edge_index.astype(jnp.int32).reshape(2, NW, n_chunks, CH)

    zrows = n_acc // NTILE
    zeros_deg = jnp.zeros((zrows, DEGW), jnp.float32)
    ones_deg = jnp.ones((CH, DEGW), jnp.float32)
    zeros_f = jnp.zeros((zrows, d_in), jnp.float32)

    deg = _degree_kernel(n_acc, n_chunks, mesh)(eidx, ones_deg, zeros_deg)

    rb = 1000
    grid1 = n // rb
    hs1, onorm, inorm = pl.pallas_call(
        _norms_hs_body,
        grid=(grid1,),
        in_specs=[
            pl.BlockSpec((NSC, 2, rb, DEGW), lambda i: (0, 0, i, 0)),
            pl.BlockSpec((rb, d_in), lambda i: (i, 0)),
        ],
        out_specs=[
            pl.BlockSpec((rb, d_in), lambda i: (i, 0)),
            pl.BlockSpec((rb, 1), lambda i: (i, 0)),
            pl.BlockSpec((rb, 1), lambda i: (i, 0)),
        ],
        out_shape=[
            jax.ShapeDtypeStruct((n, d_in), jnp.float32),
            jax.ShapeDtypeStruct((n, 1), jnp.float32),
            jax.ShapeDtypeStruct((n, 1), jnp.float32),
        ],
    )(deg, feats)

    agg1 = _edge_agg_kernel(n_acc, d_in, n_chunks, mesh)(hs1, eidx, zeros_f)

    w23 = jnp.concatenate([W2, W3], axis=1)
    hs2 = pl.pallas_call(
        _layer1_body,
        grid=(grid1,),
        in_specs=[
            pl.BlockSpec((NSC, rb, d_in), lambda i: (0, i, 0)),
            pl.BlockSpec((rb, 1), lambda i: (i, 0)),
            pl.BlockSpec((rb, 1), lambda i: (i, 0)),
            pl.BlockSpec((d_in, d_hid), lambda i: (0, 0)),
            pl.BlockSpec((1, d_hid), lambda i: (0, 0)),
            pl.BlockSpec((d_hid, 2 * d_out), lambda i: (0, 0)),
        ],
        out_specs=pl.BlockSpec((rb, 2 * d_out), lambda i: (i, 0)),
        out_shape=jax.ShapeDtypeStruct((n, 2 * d_out), jnp.float32),
    )(agg1, inorm, onorm, W1, b1.reshape(1, d_hid), w23)

    agg2 = _edge_agg_kernel(n_acc, 2 * d_out, n_chunks, mesh)(
        hs2, eidx, zeros_f[:, :2 * d_out])

    rb2 = 400
    adj = pl.pallas_call(
        _decoder_body,
        grid=(n // rb2,),
        in_specs=[
            pl.BlockSpec((NSC, n, 2 * d_out), lambda i: (0, 0, 0)),
            pl.BlockSpec((n, 1), lambda i: (0, 0)),
            pl.BlockSpec((n, d_out), lambda i: (0, 0)),
            pl.BlockSpec((1, d_out), lambda i: (0, 0)),
            pl.BlockSpec((1, d_out), lambda i: (0, 0)),
        ],
        out_specs=pl.BlockSpec((rb2, n), lambda i: (i, 0)),
        out_shape=jax.ShapeDtypeStruct((n, n), jnp.float32),
        scratch_shapes=[pltpu.VMEM((n, d_out), jnp.float32)],
    )(agg2, inorm, noise, b2.reshape(1, d_out), b3.reshape(1, d_out))
    return adj
